# Initial kernel scaffold; baseline (speedup 1.0000x reference)
import jax
import jax.numpy as jnp
from jax.experimental import pallas as pl


def kernel(x, edge_index, W_in, b_in, g_in, be_in, W1, b1, g1, be1, W2, b2, g2, be2, W3, b3, g3, be3, W4, b4, g4, be4, W5, b5, g5, be5, Wh1, bh1, lg1, lb1, Wh2, bh2, lg2, lb2, Wh3, bh3, Wh4, bh4):
    raise NotImplementedError("write your pallas kernel here")


# speedup vs baseline: 4.7828x; 4.7828x over previous
import dataclasses
import functools

import jax
import jax.numpy as jnp
from jax import lax
from jax.experimental import pallas as pl
from jax.experimental.pallas import tpu as pltpu
from jax.experimental.pallas import tpu_sc as plsc

N = 10000
E = 320000
NTILES = 16
NW = 2 * NTILES
CH = 80
NCHD = (E // NW) // CH
KFIRE = 5
NP = 10240
NPT = NP // NTILES
RANGE = 3840
RPT = RANGE // NTILES
F32 = jnp.float32


def _sc_params():
    cp = pltpu.CompilerParams()
    if "needs_layout_passes" in pltpu.CompilerParams.__dataclass_fields__:
        cp = dataclasses.replace(cp, needs_layout_passes=False)
    return cp


@functools.cache
def _make_agg():
    @functools.partial(
        pl.kernel,
        mesh=plsc.VectorSubcoreMesh(core_axis_name="c", subcore_axis_name="s"),
        compiler_params=_sc_params(),
        out_type=[jax.ShapeDtypeStruct((2, RANGE, 128), F32) for _ in range(2)],
        scratch_types=[
            pltpu.VMEM((NCHD, CH), jnp.int32),
            pltpu.VMEM((NCHD, CH), jnp.int32),
            *[pltpu.VMEM((CH, 128), F32) for _ in range(KFIRE)],
            pltpu.VMEM((CH, 128), F32),
            pltpu.VMEM_SHARED((RANGE, 128), F32),
            pltpu.VMEM((16,), jnp.int32),
            pltpu.VMEM((16,), jnp.int32),
            pltpu.SemaphoreType.DMA,
            pltpu.SemaphoreType.DMA,
        ],
    )
    def agg(tables_hbm, edge_hbm, flag_hbm, base_hbm, out0, out1,
            src_v, dst_v, b0, b1, b2, b3, b4, z_v, acc, flag_v, base_v,
            sem_g, sem_s):
        bufs = [b0, b1, b2, b3, b4]
        c = lax.axis_index("c")
        s = lax.axis_index("s")
        w = c * NTILES + s
        pltpu.sync_copy(edge_hbm.at[0, w], src_v)
        pltpu.sync_copy(edge_hbm.at[1, w], dst_v)
        pltpu.sync_copy(flag_hbm, flag_v)
        pltpu.sync_copy(base_hbm, base_v)
        base = jnp.max(base_v[...])

        @pl.loop(0, NCHD)
        def _(j):
            for k in range(CH // 16):
                sl = (j, pl.ds(16 * k, 16))
                sv = src_v[sl]
                dv = dst_v[sl]
                r = dv - base
                inr = (dv >= base) & (r < RANGE)
                src_v[sl] = jnp.where(inr, sv, N + (sv & 127))
                dst_v[sl] = jnp.where(inr, r, dv & 2047)

        @pl.loop(0, CH)
        def _(i):
            @pl.loop(0, 128, step=16)
            def _(j):
                z_v[i, pl.ds(j, 16)] = jnp.zeros((16,), F32)

        npages = 1 + jnp.max(flag_v[...])

        @pl.loop(0, npages)
        def _(p):
            off = pl.multiple_of(p * NP, NP)
            table = tables_hbm.at[pl.ds(off, NP)]
            for i in range(3):
                pltpu.sync_copy(z_v, acc.at[pl.ds(s * RPT + i * CH, CH)])

            plsc.subcore_barrier()

            @pl.loop(0, NCHD, step=KFIRE)
            def _(j0):
                gs = [pltpu.async_copy(table.at[src_v.at[j0 + k]], bufs[k],
                                       sem_g) for k in range(KFIRE)]
                ss = []
                for k in range(KFIRE):
                    gs[k].wait()
                    ss.append(pltpu.async_copy(bufs[k],
                                               acc.at[dst_v.at[j0 + k]],
                                               sem_s, add=True))
                for cp in ss:
                    cp.wait()

            plsc.subcore_barrier()

            @pl.when(c == 0)
            def _():
                pltpu.sync_copy(acc.at[pl.ds(s * RPT, RPT)],
                                out0.at[p, pl.ds(s * RPT, RPT)])

            @pl.when(c == 1)
            def _():
                pltpu.sync_copy(acc.at[pl.ds(s * RPT, RPT)],
                                out1.at[p, pl.ds(s * RPT, RPT)])

            plsc.subcore_barrier()

    return agg


@functools.cache
def _make_deg():
    @functools.partial(
        pl.kernel,
        mesh=plsc.VectorSubcoreMesh(core_axis_name="c", subcore_axis_name="s"),
        compiler_params=_sc_params(),
        out_type=jax.ShapeDtypeStruct((2, NP, 16), F32),
        scratch_types=[
            pltpu.VMEM((NCHD, CH), jnp.int32),
            pltpu.VMEM((CH, 16), F32),
            pltpu.VMEM((NPT // 4, 16), F32),
            pltpu.VMEM_SHARED((NP, 16), F32),
            pltpu.SemaphoreType.DMA,
        ],
    )
    def deg(edge_hbm, out_hbm, dst_v, ones_v, z_v, acc, sem):
        c = lax.axis_index("c")
        s = lax.axis_index("s")
        pltpu.sync_copy(edge_hbm.at[1, c * NTILES + s], dst_v)

        @pl.loop(0, CH)
        def _(i):
            ones_v[i, :] = jnp.full((16,), 1.0, F32)

        @pl.loop(0, NPT // 4)
        def _(i):
            z_v[i, :] = jnp.full((16,), 0.0, F32)

        for i in range(4):
            pltpu.sync_copy(
                z_v, acc.at[pl.ds(s * NPT + i * (NPT // 4), NPT // 4)])
        plsc.subcore_barrier()

        @pl.loop(0, NCHD)
        def _(j):
            pltpu.sync_copy(ones_v, acc.at[dst_v.at[j]], add=True)

        plsc.subcore_barrier()
        pltpu.sync_copy(acc.at[pl.ds(s * NPT, NPT)],
                        out_hbm.at[c, pl.ds(s * NPT, NPT)])

    return deg


def _bn(h, g, b):
    m = jnp.mean(h, axis=0, keepdims=True)
    v = jnp.mean((h - m) ** 2, axis=0, keepdims=True)
    return (h - m) * lax.rsqrt(v + 1e-5) * g + b


def _ln(h, g, b):
    m = jnp.mean(h, axis=-1, keepdims=True)
    v = jnp.mean((h - m) ** 2, axis=-1, keepdims=True)
    return (h - m) * lax.rsqrt(v + 1e-5) * g + b


def _h0_body(x_ref, w_ref, b_ref, g_ref, be_ref, o_ref):
    h = jnp.dot(x_ref[...], w_ref[...], preferred_element_type=F32) + b_ref[...]
    o_ref[...] = jnp.maximum(_bn(h, g_ref[...], be_ref[...]), 0.0)


def _tables_store(h, o_ref):
    o_ref[:N, :] = h[:, :128]
    o_ref[N:NP, :] = jnp.zeros((NP - N, 128), F32)
    o_ref[NP:NP + N, :] = h[:, 128:]
    o_ref[NP + N:, :] = jnp.zeros((NP - N, 128), F32)


def _pre1_body(h0_ref, degp_ref, w_ref, dinv_ref, o_ref):
    deg = degp_ref[0, :N, 0:1] + degp_ref[1, :N, 0:1] + 1.0
    dinv = lax.rsqrt(deg)
    dinv_ref[...] = dinv
    h = jnp.dot(h0_ref[...], w_ref[...], preferred_element_type=F32) * dinv
    _tables_store(h, o_ref)


def _pre_body(h_ref, dinv_ref, w_ref, o_ref):
    h = jnp.dot(h_ref[...], w_ref[...], preferred_element_type=F32) * dinv_ref[...]
    _tables_store(h, o_ref)


def _psum0_body(a0_ref, b0_ref, a1_ref, b1_ref, a2_ref, b2_ref, t_ref, o_ref):
    o_ref[:RANGE, :] = a0_ref[...] + b0_ref[...] + t_ref[:RANGE, :]
    o_ref[RANGE:2 * RANGE, :] = (a1_ref[...] + b1_ref[...]
                                 + t_ref[RANGE:2 * RANGE, :])
    o_ref[2 * RANGE:, :] = (a2_ref[:N - 2 * RANGE, :]
                            + b2_ref[:N - 2 * RANGE, :]
                            + t_ref[2 * RANGE:N, :])


def _psum1_body(a0_ref, b0_ref, a1_ref, b1_ref, a2_ref, b2_ref, t_ref,
                flag_ref, o_ref):
    f = flag_ref[...] != 0.0
    o_ref[:RANGE, :] = jnp.where(
        f, a0_ref[...] + b0_ref[...] + t_ref[NP:NP + RANGE, :], 0.0)
    o_ref[RANGE:2 * RANGE, :] = jnp.where(
        f, a1_ref[...] + b1_ref[...] + t_ref[NP + RANGE:NP + 2 * RANGE, :], 0.0)
    o_ref[2 * RANGE:, :] = jnp.where(
        f, a2_ref[:N - 2 * RANGE, :] + b2_ref[:N - 2 * RANGE, :]
        + t_ref[NP + 2 * RANGE:NP + N, :], 0.0)


def _post_body(s0_ref, s1_ref, dinv_ref, b_ref, g_ref, be_ref, r_ref,
               hp_ref, o_ref):
    agg = jnp.concatenate([s0_ref[...], s1_ref[...]], axis=1)
    out = agg * dinv_ref[...] + b_ref[...]
    o_ref[...] = (jnp.maximum(_bn(out, g_ref[...], be_ref[...]), 0.0)
                  + r_ref[...] * hp_ref[...])


def _head_body(h5_ref,
               wh1_ref, bh1_ref, lg1_ref, lb1_ref,
               wh2_ref, bh2_ref, lg2_ref, lb2_ref,
               wh3_ref, bh3_ref, wh4_ref, bh4_ref, o_ref):
    h5 = h5_ref[:, :64]
    o = jnp.dot(h5, wh1_ref[...], preferred_element_type=F32) + bh1_ref[...]
    o = jnp.maximum(_ln(o, lg1_ref[...], lb1_ref[...]), 0.0)
    o = jnp.dot(o, wh2_ref[...], preferred_element_type=F32) + bh2_ref[...]
    o = jnp.maximum(_ln(o, lg2_ref[...], lb2_ref[...]), 0.0)
    o = jnp.maximum(jnp.dot(o, wh3_ref[...], preferred_element_type=F32)
                    + bh3_ref[...], 0.0)
    o_ref[...] = jnp.dot(o, wh4_ref[...], preferred_element_type=F32) + bh4_ref[...]


def _tc_call(body, n_in, out_shapes, interpret=False):
    return pl.pallas_call(
        body,
        in_specs=[pl.BlockSpec(memory_space=pltpu.VMEM)] * n_in,
        out_specs=[pl.BlockSpec(memory_space=pltpu.VMEM)] * len(out_shapes),
        out_shape=[jax.ShapeDtypeStruct(s, F32) for s in out_shapes],
        interpret=interpret,
    )


def _pad_to(a, shape):
    pads = [(0, t - s) for s, t in zip(a.shape, shape)]
    return jnp.pad(a, pads)


def kernel(x, edge_index, W_in, b_in, g_in, be_in, W1, b1, g1, be1, W2, b2,
           g2, be2, W3, b3, g3, be3, W4, b4, g4, be4, W5, b5, g5, be5, Wh1,
           bh1, lg1, lb1, Wh2, bh2, lg2, lb2, Wh3, bh3, Wh4, bh4):
    edge_d = edge_index.reshape(2, NW, NCHD, CH)
    row = lambda v: v.reshape(1, -1)
    agg = _make_agg()

    wnext = jnp.stack([
        _pad_to(W2.T, (256, 256)), _pad_to(W3.T, (256, 256)),
        _pad_to(W4.T, (256, 256)), _pad_to(W5.T, (256, 256)),
        jnp.zeros((256, 256), F32),
    ])
    bs = jnp.stack([_pad_to(row(b), (1, 256)) for b in (b1, b2, b3, b4, b5)])
    gs = jnp.stack([_pad_to(row(g), (1, 256)) for g in (g1, g2, g3, g4, g5)])
    bes = jnp.stack([_pad_to(row(b), (1, 256)) for b in (be1, be2, be3, be4, be5)])
    rmask = jnp.array([0.0, 1.0, 0.0, 1.0, 0.0], F32).reshape(5, 1, 1)
    flags_i = jnp.repeat(jnp.array([1, 1, 0, 0, 0], jnp.int32)[:, None], 16, axis=1)
    flags_f = flags_i[:, :1].astype(F32).reshape(5, 1, 1)

    degp = _make_deg()(edge_d)
    h0, = _tc_call(_h0_body, 5, [(N, 256)])(
        x, W_in.T, row(b_in), row(g_in), row(be_in))
    dinv, tables = _tc_call(_pre1_body, 3, [(N, 1), (2 * NP, 128)])(
        h0, degp, _pad_to(W1.T, (256, 256)))

    psum0 = _tc_call(_psum0_body, 7, [(N, 128)])
    psum1 = _tc_call(_psum1_body, 8, [(N, 128)])
    bases = [jnp.full((16,), i * RANGE, jnp.int32) for i in range(3)]
    post = _tc_call(_post_body, 8, [(N, 256)])
    pre = _tc_call(_pre_body, 3, [(2 * NP, 128)])

    def body(carry, xs):
        h_prev, tables = carry
        b, g, be, r, fi, ff, wn = xs
        parts = []
        chain = tables
        for i in range(3):
            p0i, p1i = agg(chain, edge_d, fi, bases[i])
            parts.append((p0i, p1i))
            if i < 2:
                chain, _ = lax.optimization_barrier((chain, p0i))
        s0, = psum0(parts[0][0][0], parts[0][1][0], parts[1][0][0],
                    parts[1][1][0], parts[2][0][0], parts[2][1][0], tables)
        s1, = psum1(parts[0][0][1], parts[0][1][1], parts[1][0][1],
                    parts[1][1][1], parts[2][0][1], parts[2][1][1], tables, ff)
        h_new, = post(s0, s1, dinv, b, g, be, r, h_prev)
        tables_new, = pre(h_new, dinv, wn)
        return (h_new, tables_new), None

    (h5, _), _ = lax.scan(
        body, (h0, tables), (bs, gs, bes, rmask, flags_i, flags_f, wnext))

    o, = _tc_call(_head_body, 13, [(N, 1)])(
        h5,
        Wh1.T, row(bh1), row(lg1), row(lb1),
        Wh2.T, row(bh2), row(lg2), row(lb2),
        Wh3.T, row(bh3), Wh4.T, row(bh4))
    return o[:, 0]
